# Initial kernel scaffold; baseline (speedup 1.0000x reference)
import jax
import jax.numpy as jnp
from jax.experimental import pallas as pl


def kernel(x, W1, att_src1, att_dst1, b1, W2, att_src2, att_dst2, b2):
    raise NotImplementedError("write your pallas kernel here")


# speedup vs baseline: 3310.9946x; 3310.9946x over previous
import jax
import jax.numpy as jnp
from jax import lax
from jax.experimental import pallas as pl

N = 512
D = 128
H1 = 4
C1 = 64
HID = H1 * C1
C2 = 128


def _rowmax(m):
    return jnp.max(m, axis=1, keepdims=True)


def _gat_dense(h, a_srcT, a_dst):
    e = a_dst + a_srcT
    e = jnp.where(e >= 0.0, e, 0.2 * e)
    p = jnp.exp(e - _rowmax(e))
    denom = jnp.sum(p, axis=1, keepdims=True)
    out = jnp.dot(p, h, preferred_element_type=jnp.float32)
    return out / (denom + 1e-16)


def _encoder_kernel(x_ref, W1_ref, As1_ref, Ad1_ref, b1_ref,
                    W2_ref, as2_ref, ad2_ref, b2_ref, out_ref):
    x = x_ref[...]
    h1 = jnp.dot(x, W1_ref[...], preferred_element_type=jnp.float32)

    a_dst = jnp.dot(h1, Ad1_ref[...], preferred_element_type=jnp.float32)
    a_srcT = lax.dot_general(As1_ref[...], h1,
                             (((0,), (1,)), ((), ())),
                             preferred_element_type=jnp.float32)

    outs = []
    for hd in range(H1):
        h_head = h1[:, hd * C1:(hd + 1) * C1]
        outs.append(_gat_dense(h_head,
                               a_srcT[hd:hd + 1, :],
                               a_dst[:, hd:hd + 1]))
    o1 = jnp.concatenate(outs, axis=1) + b1_ref[...]
    o1 = jnp.maximum(o1, 0.0)

    h2 = jnp.dot(o1, W2_ref[...], preferred_element_type=jnp.float32)
    a_dst2 = jnp.dot(h2, ad2_ref[...], preferred_element_type=jnp.float32)
    a_src2T = lax.dot_general(as2_ref[...], h2,
                              (((0,), (1,)), ((), ())),
                              preferred_element_type=jnp.float32)
    o2 = _gat_dense(h2, a_src2T, a_dst2) + b2_ref[...]
    out_ref[...] = o2


def kernel(x, W1, att_src1, att_dst1, b1, W2, att_src2, att_dst2, b2):
    def blockdiag(att):
        _, h, c = att.shape
        eye = jnp.eye(h, dtype=att.dtype)
        return (att[0][:, :, None] * eye[:, None, :]).reshape(h * c, h)

    As1 = blockdiag(att_src1)
    Ad1 = blockdiag(att_dst1)
    as2 = att_src2.reshape(C2, 1)
    ad2 = att_dst2.reshape(C2, 1)

    return pl.pallas_call(
        _encoder_kernel,
        out_shape=jax.ShapeDtypeStruct((N, C2), jnp.float32),
    )(x.astype(jnp.float32), W1, As1, Ad1, b1.reshape(1, HID),
      W2, as2, ad2, b2.reshape(1, C2))


# speedup vs baseline: 3841.6179x; 1.1603x over previous
import jax
import jax.numpy as jnp
from jax import lax
from jax.experimental import pallas as pl

N = 512
D = 128
H1 = 4
C1 = 64
HID = H1 * C1
C2 = 128


def _rowmax(m):
    return jnp.max(m, axis=1, keepdims=True)


def _gat_dense(h_ext, a_srcT, a_dst):
    e = a_dst + a_srcT
    e = jnp.maximum(e, 0.2 * e)
    p = jnp.exp(e - _rowmax(e))
    acc = jnp.dot(p, h_ext, preferred_element_type=jnp.float32)
    c = h_ext.shape[1] - 1
    return acc[:, :c] / (acc[:, c:] + 1e-16)


def _encoder_kernel(x_ref, W1_ref, As1_ref, Ad1_ref, b1_ref,
                    W2_ref, as2_ref, ad2_ref, b2_ref, out_ref):
    x = x_ref[...]
    h1 = jnp.dot(x, W1_ref[...], preferred_element_type=jnp.float32)

    a_dst = jnp.dot(h1, Ad1_ref[...], preferred_element_type=jnp.float32)
    a_srcT = lax.dot_general(As1_ref[...], h1,
                             (((0,), (1,)), ((), ())),
                             preferred_element_type=jnp.float32)

    ones = jnp.ones((N, 1), dtype=jnp.float32)
    outs = []
    for hd in range(H1):
        h_ext = jnp.concatenate([h1[:, hd * C1:(hd + 1) * C1], ones], axis=1)
        outs.append(_gat_dense(h_ext,
                               a_srcT[hd:hd + 1, :],
                               a_dst[:, hd:hd + 1]))
    o1 = jnp.concatenate(outs, axis=1) + b1_ref[...]
    o1 = jnp.maximum(o1, 0.0)

    h2 = jnp.dot(o1, W2_ref[...], preferred_element_type=jnp.float32)
    a_dst2 = jnp.dot(h2, ad2_ref[...], preferred_element_type=jnp.float32)
    a_src2T = lax.dot_general(as2_ref[...], h2,
                              (((0,), (1,)), ((), ())),
                              preferred_element_type=jnp.float32)
    h2_ext = jnp.concatenate([h2, ones], axis=1)
    o2 = _gat_dense(h2_ext, a_src2T, a_dst2) + b2_ref[...]
    out_ref[...] = o2


def kernel(x, W1, att_src1, att_dst1, b1, W2, att_src2, att_dst2, b2):
    def blockdiag(att):
        _, h, c = att.shape
        eye = jnp.eye(h, dtype=att.dtype)
        return (att[0][:, :, None] * eye[:, None, :]).reshape(h * c, h)

    As1 = blockdiag(att_src1)
    Ad1 = blockdiag(att_dst1)
    as2 = att_src2.reshape(C2, 1)
    ad2 = att_dst2.reshape(C2, 1)

    return pl.pallas_call(
        _encoder_kernel,
        out_shape=jax.ShapeDtypeStruct((N, C2), jnp.float32),
    )(x.astype(jnp.float32), W1, As1, Ad1, b1.reshape(1, HID),
      W2, as2, ad2, b2.reshape(1, C2))


# speedup vs baseline: 4886.8071x; 1.2721x over previous
import jax
import jax.numpy as jnp
from jax import lax
from jax.experimental import pallas as pl

N = 512
D = 128
H1 = 4
C1 = 64
HID = H1 * C1
C2 = 128


def _rowmax(m):
    return jnp.max(m, axis=1, keepdims=True)


def _gat_dense(h_ext, a_srcT, a_dst):
    e = a_dst + a_srcT
    e = jnp.maximum(e, 0.2 * e)
    p = jnp.exp(e - _rowmax(e))
    acc = jnp.dot(p, h_ext, preferred_element_type=jnp.float32)
    c = h_ext.shape[1] - 1
    return acc[:, :c] / (acc[:, c:] + 1e-16)


def _blockdiag(att_row, heads, ch):
    att_col = jnp.transpose(att_row)
    if heads == 1:
        return att_col
    r = lax.broadcasted_iota(jnp.int32, (heads * ch, heads), 0) // ch
    c = lax.broadcasted_iota(jnp.int32, (heads * ch, heads), 1)
    return jnp.where(r == c, att_col, 0.0)


def _encoder_kernel(x_ref, W1_ref, as1_ref, ad1_ref, b1_ref,
                    W2_ref, as2_ref, ad2_ref, b2_ref, out_ref):
    x = x_ref[...]
    h1 = jnp.dot(x, W1_ref[...], preferred_element_type=jnp.float32)

    As1 = _blockdiag(as1_ref[...], H1, C1)
    Ad1 = _blockdiag(ad1_ref[...], H1, C1)
    a_dst = jnp.dot(h1, Ad1, preferred_element_type=jnp.float32)
    a_srcT = lax.dot_general(As1, h1, (((0,), (1,)), ((), ())),
                             preferred_element_type=jnp.float32)

    ones = jnp.ones((N, 1), dtype=jnp.float32)
    outs = []
    for hd in range(H1):
        h_ext = jnp.concatenate([h1[:, hd * C1:(hd + 1) * C1], ones], axis=1)
        outs.append(_gat_dense(h_ext,
                               a_srcT[hd:hd + 1, :],
                               a_dst[:, hd:hd + 1]))
    o1 = jnp.concatenate(outs, axis=1) + b1_ref[...]
    o1 = jnp.maximum(o1, 0.0)

    h2 = jnp.dot(o1, W2_ref[...], preferred_element_type=jnp.float32)
    ad2_col = _blockdiag(ad2_ref[...], 1, C2)
    as2_col = _blockdiag(as2_ref[...], 1, C2)
    a_dst2 = jnp.dot(h2, ad2_col, preferred_element_type=jnp.float32)
    a_src2T = lax.dot_general(as2_col, h2, (((0,), (1,)), ((), ())),
                              preferred_element_type=jnp.float32)
    h2_ext = jnp.concatenate([h2, ones], axis=1)
    o2 = _gat_dense(h2_ext, a_src2T, a_dst2) + b2_ref[...]
    out_ref[...] = o2


def kernel(x, W1, att_src1, att_dst1, b1, W2, att_src2, att_dst2, b2):
    return pl.pallas_call(
        _encoder_kernel,
        out_shape=jax.ShapeDtypeStruct((N, C2), jnp.float32),
    )(x, W1,
      att_src1.reshape(1, HID), att_dst1.reshape(1, HID), b1.reshape(1, HID),
      W2,
      att_src2.reshape(1, C2), att_dst2.reshape(1, C2), b2.reshape(1, C2))


# speedup vs baseline: 5265.5707x; 1.0775x over previous
import jax
import jax.numpy as jnp
from jax import lax
from jax.experimental import pallas as pl

N = 512
D = 128
H1 = 4
C1 = 64
HID = H1 * C1
C2 = 128


def _gat_dense(h_ext, a_srcT, a_dst):
    s_max = jnp.max(a_srcT)
    dps = a_dst + s_max
    m = jnp.maximum(dps, 0.2 * dps)
    u = (a_dst - m) + a_srcT
    v = (0.2 * a_dst - m) + 0.2 * a_srcT
    p = jnp.exp(jnp.maximum(u, v))
    acc = jnp.dot(p, h_ext, preferred_element_type=jnp.float32)
    c = h_ext.shape[1] - 1
    return acc[:, :c] / (acc[:, c:] + 1e-16)


def _blockdiag(att_row, heads, ch):
    att_col = jnp.transpose(att_row)
    if heads == 1:
        return att_col
    r = lax.broadcasted_iota(jnp.int32, (heads * ch, heads), 0) // ch
    c = lax.broadcasted_iota(jnp.int32, (heads * ch, heads), 1)
    return jnp.where(r == c, att_col, 0.0)


def _encoder_kernel(x_ref, W1_ref, as1_ref, ad1_ref, b1_ref,
                    W2_ref, as2_ref, ad2_ref, b2_ref, out_ref):
    x = x_ref[...]
    h1 = jnp.dot(x, W1_ref[...], preferred_element_type=jnp.float32)

    As1 = _blockdiag(as1_ref[...], H1, C1)
    Ad1 = _blockdiag(ad1_ref[...], H1, C1)
    a_dst = jnp.dot(h1, Ad1, preferred_element_type=jnp.float32)
    a_srcT = lax.dot_general(As1, h1, (((0,), (1,)), ((), ())),
                             preferred_element_type=jnp.float32)

    ones = jnp.ones((N, 1), dtype=jnp.float32)
    outs = []
    for hd in range(H1):
        h_ext = jnp.concatenate([h1[:, hd * C1:(hd + 1) * C1], ones], axis=1)
        outs.append(_gat_dense(h_ext,
                               a_srcT[hd:hd + 1, :],
                               a_dst[:, hd:hd + 1]))
    o1 = jnp.concatenate(outs, axis=1) + b1_ref[...]
    o1 = jnp.maximum(o1, 0.0)

    h2 = jnp.dot(o1, W2_ref[...], preferred_element_type=jnp.float32)
    ad2_col = _blockdiag(ad2_ref[...], 1, C2)
    as2_col = _blockdiag(as2_ref[...], 1, C2)
    a_dst2 = jnp.dot(h2, ad2_col, preferred_element_type=jnp.float32)
    a_src2T = lax.dot_general(as2_col, h2, (((0,), (1,)), ((), ())),
                              preferred_element_type=jnp.float32)
    h2_ext = jnp.concatenate([h2, ones], axis=1)
    o2 = _gat_dense(h2_ext, a_src2T, a_dst2) + b2_ref[...]
    out_ref[...] = o2


def kernel(x, W1, att_src1, att_dst1, b1, W2, att_src2, att_dst2, b2):
    return pl.pallas_call(
        _encoder_kernel,
        out_shape=jax.ShapeDtypeStruct((N, C2), jnp.float32),
    )(x, W1,
      att_src1.reshape(1, HID), att_dst1.reshape(1, HID), b1.reshape(1, HID),
      W2,
      att_src2.reshape(1, C2), att_dst2.reshape(1, C2), b2.reshape(1, C2))


# speedup vs baseline: 5352.4665x; 1.0165x over previous
import jax
import jax.numpy as jnp
from jax import lax
from jax.experimental import pallas as pl

N = 512
D = 128
H1 = 4
C1 = 64
HID = H1 * C1
C2 = 128


def _gat_dense(h_ext, a_srcT, a_dst):
    s_max = jnp.max(a_srcT)
    dps = a_dst + s_max
    m = jnp.maximum(dps, 0.2 * dps)
    cu = jnp.exp(a_dst - m)
    cv = jnp.exp(0.2 * a_dst - m)
    ru = jnp.exp(a_srcT)
    rv = jnp.exp(0.2 * a_srcT)
    p = jnp.maximum(cu * ru, cv * rv)
    acc = jnp.dot(p, h_ext, preferred_element_type=jnp.float32)
    c = h_ext.shape[1] - 1
    return acc[:, :c] / (acc[:, c:] + 1e-16)


def _blockdiag(att_row, heads, ch):
    att_col = jnp.transpose(att_row)
    if heads == 1:
        return att_col
    r = lax.broadcasted_iota(jnp.int32, (heads * ch, heads), 0) // ch
    c = lax.broadcasted_iota(jnp.int32, (heads * ch, heads), 1)
    return jnp.where(r == c, att_col, 0.0)


def _encoder_kernel(x_ref, W1_ref, as1_ref, ad1_ref, b1_ref,
                    W2_ref, as2_ref, ad2_ref, b2_ref, out_ref):
    x = x_ref[...]
    h1 = jnp.dot(x, W1_ref[...], preferred_element_type=jnp.float32)

    As1 = _blockdiag(as1_ref[...], H1, C1)
    Ad1 = _blockdiag(ad1_ref[...], H1, C1)
    a_dst = jnp.dot(h1, Ad1, preferred_element_type=jnp.float32)
    a_srcT = lax.dot_general(As1, h1, (((0,), (1,)), ((), ())),
                             preferred_element_type=jnp.float32)

    ones = jnp.ones((N, 1), dtype=jnp.float32)
    outs = []
    for hd in range(H1):
        h_ext = jnp.concatenate([h1[:, hd * C1:(hd + 1) * C1], ones], axis=1)
        outs.append(_gat_dense(h_ext,
                               a_srcT[hd:hd + 1, :],
                               a_dst[:, hd:hd + 1]))
    o1 = jnp.concatenate(outs, axis=1) + b1_ref[...]
    o1 = jnp.maximum(o1, 0.0)

    h2 = jnp.dot(o1, W2_ref[...], preferred_element_type=jnp.float32)
    ad2_col = _blockdiag(ad2_ref[...], 1, C2)
    as2_col = _blockdiag(as2_ref[...], 1, C2)
    a_dst2 = jnp.dot(h2, ad2_col, preferred_element_type=jnp.float32)
    a_src2T = lax.dot_general(as2_col, h2, (((0,), (1,)), ((), ())),
                              preferred_element_type=jnp.float32)
    h2_ext = jnp.concatenate([h2, ones], axis=1)
    o2 = _gat_dense(h2_ext, a_src2T, a_dst2) + b2_ref[...]
    out_ref[...] = o2


def kernel(x, W1, att_src1, att_dst1, b1, W2, att_src2, att_dst2, b2):
    return pl.pallas_call(
        _encoder_kernel,
        out_shape=jax.ShapeDtypeStruct((N, C2), jnp.float32),
    )(x, W1,
      att_src1.reshape(1, HID), att_dst1.reshape(1, HID), b1.reshape(1, HID),
      W2,
      att_src2.reshape(1, C2), att_dst2.reshape(1, C2), b2.reshape(1, C2))


# speedup vs baseline: 5415.8248x; 1.0118x over previous
import jax
import jax.numpy as jnp
from jax import lax
from jax.experimental import pallas as pl

N = 512
D = 128
H1 = 4
C1 = 64
HID = H1 * C1
C2 = 128


def _softmax_factors(a_srcT, a_dst):
    s_max = jnp.max(a_srcT, axis=1, keepdims=True)
    dps = a_dst + jnp.transpose(s_max)
    m = jnp.maximum(dps, 0.2 * dps)
    cu = jnp.exp(a_dst - m)
    cv = jnp.exp(0.2 * a_dst - m)
    ru = jnp.exp(a_srcT)
    rv = jnp.exp(0.2 * a_srcT)
    return cu, cv, ru, rv


def _gat_dense(h_ext, cu, cv, ru, rv):
    p = jnp.maximum(cu * ru, cv * rv)
    acc = jnp.dot(p, h_ext, preferred_element_type=jnp.float32)
    c = h_ext.shape[1] - 1
    return acc[:, :c] / (acc[:, c:] + 1e-16)


def _blockdiag(att_row, heads, ch):
    att_col = jnp.transpose(att_row)
    if heads == 1:
        return att_col
    r = lax.broadcasted_iota(jnp.int32, (heads * ch, heads), 0) // ch
    c = lax.broadcasted_iota(jnp.int32, (heads * ch, heads), 1)
    return jnp.where(r == c, att_col, 0.0)


def _encoder_kernel(x_ref, W1_ref, as1_ref, ad1_ref, b1_ref,
                    W2_ref, as2_ref, ad2_ref, b2_ref, out_ref):
    x = x_ref[...]
    h1 = jnp.dot(x, W1_ref[...], preferred_element_type=jnp.float32)

    ones = jnp.ones((N, 1), dtype=jnp.float32)
    h_exts = [jnp.concatenate([h1[:, hd * C1:(hd + 1) * C1], ones], axis=1)
              for hd in range(H1)]

    As1 = _blockdiag(as1_ref[...], H1, C1)
    Ad1 = _blockdiag(ad1_ref[...], H1, C1)
    a_dst = jnp.dot(h1, Ad1, preferred_element_type=jnp.float32)
    a_srcT = lax.dot_general(As1, h1, (((0,), (1,)), ((), ())),
                             preferred_element_type=jnp.float32)

    cu, cv, ru, rv = _softmax_factors(a_srcT, a_dst)

    outs = []
    for hd in range(H1):
        outs.append(_gat_dense(h_exts[hd],
                               cu[:, hd:hd + 1], cv[:, hd:hd + 1],
                               ru[hd:hd + 1, :], rv[hd:hd + 1, :]))
    o1 = jnp.concatenate(outs, axis=1) + b1_ref[...]
    o1 = jnp.maximum(o1, 0.0)

    h2 = jnp.dot(o1, W2_ref[...], preferred_element_type=jnp.float32)
    ad2_col = _blockdiag(ad2_ref[...], 1, C2)
    as2_col = _blockdiag(as2_ref[...], 1, C2)
    a_dst2 = jnp.dot(h2, ad2_col, preferred_element_type=jnp.float32)
    a_src2T = lax.dot_general(as2_col, h2, (((0,), (1,)), ((), ())),
                              preferred_element_type=jnp.float32)
    cu2, cv2, ru2, rv2 = _softmax_factors(a_src2T, a_dst2)
    h2_ext = jnp.concatenate([h2, ones], axis=1)
    o2 = _gat_dense(h2_ext, cu2, cv2, ru2, rv2) + b2_ref[...]
    out_ref[...] = o2


def kernel(x, W1, att_src1, att_dst1, b1, W2, att_src2, att_dst2, b2):
    return pl.pallas_call(
        _encoder_kernel,
        out_shape=jax.ShapeDtypeStruct((N, C2), jnp.float32),
    )(x, W1,
      att_src1.reshape(1, HID), att_dst1.reshape(1, HID), b1.reshape(1, HID),
      W2,
      att_src2.reshape(1, C2), att_dst2.reshape(1, C2), b2.reshape(1, C2))
